# baseline (speedup 1.0000x reference)
import functools

import jax
import jax.numpy as jnp
import numpy as np
from jax.experimental import pallas as pl
from jax.experimental.pallas import tpu as pltpu

_IGNORE = 255
_THRESH = float(-np.log(0.7))


def _ohem_kernel(preds_ref, labels_ref, out_ref, loss_buf, acc, *, nsteps):
    i = pl.program_id(0)

    @pl.when(i == 0)
    def _init():
        acc[0] = 0.0
        acc[1] = 0.0
        acc[2] = 0.0

    x = preds_ref[0]
    lbl = labels_ref[0, 0]
    m = jnp.max(x, axis=0)
    e = jnp.exp(x - m[None, :])
    s = jnp.sum(e, axis=0)
    cls = jax.lax.broadcasted_iota(jnp.int32, x.shape, 0)
    xl = jnp.sum(jnp.where(cls == lbl[None, :], x, 0.0), axis=0)
    lse = jnp.log(s) + m
    valid = lbl != _IGNORE
    loss = jnp.where(valid, lse - xl, 0.0)
    loss_buf[pl.ds(i, 1), :] = loss[None, :]
    hard = loss > _THRESH
    acc[0] = acc[0] + jnp.sum(valid.astype(jnp.float32))
    acc[1] = acc[1] + jnp.sum(hard.astype(jnp.float32))
    acc[2] = acc[2] + jnp.sum(jnp.where(hard, loss, 0.0))

    @pl.when(i == nsteps - 1)
    def _finish():
        cv = acc[0].astype(jnp.int32)
        nh = acc[1].astype(jnp.int32)
        n_min = cv // 16
        mean_hard = acc[2] / acc[1]

        def topk_mean():
            lb = loss_buf[...]
            bits = jax.lax.bitcast_convert_type(lb, jnp.int32)
            k = n_min

            def body(j, prefix):
                cand = prefix | (jnp.int32(1) << (30 - j))
                cnt = jnp.sum((bits >= cand).astype(jnp.int32))
                return jnp.where(cnt >= k, cand, prefix)

            vbits = jax.lax.fori_loop(0, 31, body, jnp.int32(0))
            v = jax.lax.bitcast_convert_type(vbits, jnp.float32)
            gt = bits > vbits
            cnt_gt = jnp.sum(gt.astype(jnp.int32))
            sum_gt = jnp.sum(jnp.where(gt, lb, 0.0))
            kf = k.astype(jnp.float32)
            return (sum_gt + (kf - cnt_gt.astype(jnp.float32)) * v) / kf

        out_ref[0] = jax.lax.cond(nh < n_min, topk_mean, lambda: mean_hard)


def kernel(preds, labels):
    B, C, H, W = preds.shape
    hw = H * W
    pblk = 2048 if hw % 2048 == 0 else hw
    nb = hw // pblk
    nsteps = B * nb
    preds3 = preds.reshape(B, C, hw)
    labels3 = labels.reshape(B * nb, 1, pblk)
    out = pl.pallas_call(
        functools.partial(_ohem_kernel, nsteps=nsteps),
        grid=(nsteps,),
        in_specs=[
            pl.BlockSpec((1, C, pblk), lambda i, nb=nb: (i // nb, 0, i % nb)),
            pl.BlockSpec((1, 1, pblk), lambda i: (i, 0, 0)),
        ],
        out_specs=pl.BlockSpec(memory_space=pltpu.SMEM),
        out_shape=jax.ShapeDtypeStruct((1,), jnp.float32),
        scratch_shapes=[
            pltpu.VMEM((nsteps, pblk), jnp.float32),
            pltpu.SMEM((3,), jnp.float32),
        ],
    )(preds3, labels3)
    return out[0]


# speedup vs baseline: 9.5017x; 9.5017x over previous
import functools

import jax
import jax.numpy as jnp
import numpy as np
from jax.experimental import pallas as pl
from jax.experimental.pallas import tpu as pltpu

_IGNORE = 255
_THRESH = float(-np.log(0.7))


def _ohem_kernel(preds_ref, labels_ref, out_ref, loss_buf, m_s, s_s, xl_s,
                 acc, *, cblk, nbatch, nchunk):
    b = pl.program_id(0)
    c = pl.program_id(1)

    @pl.when(jnp.logical_and(b == 0, c == 0))
    def _init_acc():
        acc[0] = 0.0
        acc[1] = 0.0
        acc[2] = 0.0

    @pl.when(c == 0)
    def _init_state():
        m_s[...] = jnp.full(m_s.shape, -jnp.inf, jnp.float32)
        s_s[...] = jnp.zeros(s_s.shape, jnp.float32)
        xl_s[...] = jnp.zeros(xl_s.shape, jnp.float32)

    x = preds_ref[0]
    lbl = labels_ref[0]
    m_old = m_s[...]
    m_new = jnp.maximum(m_old, jnp.max(x, axis=0))
    e = jnp.exp(x - m_new[None])
    csum = jnp.sum(e, axis=0)
    cls = jax.lax.broadcasted_iota(jnp.int32, x.shape, 0) + c * cblk
    cxl = jnp.sum(jnp.where(cls == lbl[None], x, 0.0), axis=0)
    s_s[...] = s_s[...] * jnp.exp(m_old - m_new) + csum
    xl_s[...] = xl_s[...] + cxl
    m_s[...] = m_new

    @pl.when(c == nchunk - 1)
    def _finish_batch():
        lse = m_s[...] + jnp.log(s_s[...])
        valid = lbl != _IGNORE
        loss = jnp.where(valid, lse - xl_s[...], 0.0)
        loss_buf[b] = loss
        hard = loss > _THRESH
        acc[0] = acc[0] + jnp.sum(valid.astype(jnp.float32))
        acc[1] = acc[1] + jnp.sum(hard.astype(jnp.float32))
        acc[2] = acc[2] + jnp.sum(jnp.where(hard, loss, 0.0))

        @pl.when(b == nbatch - 1)
        def _finish():
            cv = acc[0].astype(jnp.int32)
            nh = acc[1].astype(jnp.int32)
            n_min = cv // 16
            mean_hard = acc[2] / acc[1]

            def topk_mean():
                lb = loss_buf[...]
                bits = jax.lax.bitcast_convert_type(lb, jnp.int32)
                k = n_min

                def body(j, prefix):
                    cand = prefix | (jnp.int32(1) << (30 - j))
                    cnt = jnp.sum((bits >= cand).astype(jnp.int32))
                    return jnp.where(cnt >= k, cand, prefix)

                vbits = jax.lax.fori_loop(0, 31, body, jnp.int32(0))
                v = jax.lax.bitcast_convert_type(vbits, jnp.float32)
                gt = bits > vbits
                cnt_gt = jnp.sum(gt.astype(jnp.int32))
                sum_gt = jnp.sum(jnp.where(gt, lb, 0.0))
                kf = k.astype(jnp.float32)
                return (sum_gt + (kf - cnt_gt.astype(jnp.float32)) * v) / kf

            out_ref[0] = jax.lax.cond(nh < n_min, topk_mean,
                                      lambda: mean_hard)


def kernel(preds, labels):
    B, C, H, W = preds.shape
    hw = H * W
    L = 2048 if hw % 2048 == 0 else hw
    R = hw // L
    cblk = 15 if C % 15 == 0 else C
    nchunk = C // cblk
    preds4 = preds.reshape(B, C, R, L)
    labels3 = labels.reshape(B, R, L)
    out = pl.pallas_call(
        functools.partial(_ohem_kernel, cblk=cblk, nbatch=B, nchunk=nchunk),
        grid=(B, nchunk),
        in_specs=[
            pl.BlockSpec((1, cblk, R, L), lambda b, c: (b, c, 0, 0)),
            pl.BlockSpec((1, R, L), lambda b, c: (b, 0, 0)),
        ],
        out_specs=pl.BlockSpec(memory_space=pltpu.SMEM),
        out_shape=jax.ShapeDtypeStruct((1,), jnp.float32),
        scratch_shapes=[
            pltpu.VMEM((B, R, L), jnp.float32),
            pltpu.VMEM((R, L), jnp.float32),
            pltpu.VMEM((R, L), jnp.float32),
            pltpu.VMEM((R, L), jnp.float32),
            pltpu.SMEM((3,), jnp.float32),
        ],
    )(preds4, labels3)
    return out[0]
